# baseline (speedup 1.0000x reference)
import functools

import jax
import jax.numpy as jnp
from jax import lax
from jax.experimental import pallas as pl
from jax.experimental.pallas import tpu as pltpu
from jax.experimental.pallas import tpu_sc as plsc

_F = 4096
_NW = 32
_CI = 4
_CO = 8
_L = 16


def _build_permute(n_rows):
    rows_per_w = n_rows // _NW
    nog = rows_per_w // _CO
    nic = rows_per_w // _CI
    mesh = plsc.VectorSubcoreMesh(core_axis_name="c", subcore_axis_name="s")

    @functools.partial(
        pl.kernel,
        mesh=mesh,
        out_type=jax.ShapeDtypeStruct((n_rows, _F), jnp.float32),
        compiler_params=pltpu.CompilerParams(
            needs_layout_passes=False,
        ),
        scratch_types=[
            pltpu.VMEM((_F,), jnp.int32),
            pltpu.VMEM((_CI, _F), jnp.float32),
            pltpu.VMEM((_CI, _F), jnp.float32),
            pltpu.VMEM((_CO, _F), jnp.float32),
            pltpu.VMEM((_CO, _F), jnp.float32),
            pltpu.SemaphoreType.DMA,
            pltpu.SemaphoreType.DMA,
            pltpu.SemaphoreType.DMA,
            pltpu.SemaphoreType.DMA,
        ],
    )
    def k(x_hbm, perm_hbm, out_hbm, perm_v, i0, i1, o0, o1, si0, si1, so0, so1):
        ins, outs = (i0, i1), (o0, o1)
        sis, sos = (si0, si1), (so0, so1)
        wid = lax.axis_index("s") * 2 + lax.axis_index("c")
        pltpu.sync_copy(perm_hbm, perm_v)
        base = wid * rows_per_w

        def in_slice(ic):
            return x_hbm.at[pl.ds(base + ic * _CI, _CI)]

        def out_slice(og):
            return out_hbm.at[pl.ds(base + og * _CO, _CO)]

        pltpu.async_copy(in_slice(0), i0, si0)
        pltpu.async_copy(in_slice(1), i1, si1)

        @pl.loop(0, nog, step=2)
        def group(g):
            for b in range(2):
                og = g + b
                out_v = outs[b]

                @pl.when(og >= 2)
                def _wait_out():
                    pltpu.make_async_copy(out_v, out_slice(og), sos[b]).wait()

                for h in range(2):
                    ic = 2 * og + h
                    in_v = ins[h]
                    pltpu.make_async_copy(in_slice(ic), in_v, sis[h]).wait()

                    @plsc.parallel_loop(0, _F // _L, unroll=16)
                    def col(j):
                        pcol = perm_v[pl.ds(j * _L, _L)]
                        for r in range(_CI):
                            rvec = jnp.full((_L,), r, jnp.int32)
                            out_v[_CI * h + r, pl.ds(j * _L, _L)] = (
                                plsc.load_gather(in_v, [rvec, pcol])
                            )

                    @pl.when(ic + 2 < nic)
                    def _next_in():
                        pltpu.async_copy(in_slice(ic + 2), in_v, sis[h])

                pltpu.async_copy(out_v, out_slice(og), sos[b])

        pltpu.make_async_copy(o0, out_slice(0), so0).wait()
        pltpu.make_async_copy(o1, out_slice(1), so1).wait()

    return k


_PERMUTE = _build_permute(4 * 4096)


def kernel(x, perm):
    b, s, f = x.shape
    out = _PERMUTE(x.reshape(b * s, f), perm.astype(jnp.int32))
    return out.reshape(b, s, f)


# speedup vs baseline: 1.0121x; 1.0121x over previous
import functools

import jax
import jax.numpy as jnp
from jax import lax
from jax.experimental import pallas as pl
from jax.experimental.pallas import tpu as pltpu
from jax.experimental.pallas import tpu_sc as plsc

_F = 4096
_NW = 32
_CI = 4
_CO = 8
_L = 16


def _build_permute(n_rows):
    rows_per_w = n_rows // _NW
    nog = rows_per_w // _CO
    nic = rows_per_w // _CI
    mesh = plsc.VectorSubcoreMesh(core_axis_name="c", subcore_axis_name="s")

    @functools.partial(
        pl.kernel,
        mesh=mesh,
        out_type=jax.ShapeDtypeStruct((n_rows, _F), jnp.float32),
        compiler_params=pltpu.CompilerParams(
            needs_layout_passes=False,
        ),
        scratch_types=[
            pltpu.VMEM((_F,), jnp.int32),
            pltpu.VMEM((_CI, _F), jnp.float32),
            pltpu.VMEM((_CI, _F), jnp.float32),
            pltpu.VMEM((_CO, _F), jnp.float32),
            pltpu.VMEM((_CO, _F), jnp.float32),
            pltpu.SemaphoreType.DMA,
            pltpu.SemaphoreType.DMA,
            pltpu.SemaphoreType.DMA,
            pltpu.SemaphoreType.DMA,
        ],
    )
    def k(x_hbm, perm_hbm, out_hbm, perm_v, i0, i1, o0, o1, si0, si1, so0, so1):
        ins, outs = (i0, i1), (o0, o1)
        sis, sos = (si0, si1), (so0, so1)
        wid = lax.axis_index("s") * 2 + lax.axis_index("c")
        pltpu.sync_copy(perm_hbm, perm_v)

        def in_slice(ic):
            og, h = ic // 2, ic % 2
            return x_hbm.at[pl.ds((wid + og * _NW) * _CO + h * _CI, _CI)]

        def out_slice(og):
            return out_hbm.at[pl.ds((wid + og * _NW) * _CO, _CO)]

        pltpu.async_copy(in_slice(0), i0, si0)
        pltpu.async_copy(in_slice(1), i1, si1)

        @pl.loop(0, nog, step=2)
        def group(g):
            for b in range(2):
                og = g + b
                out_v = outs[b]

                @pl.when(og >= 2)
                def _wait_out():
                    pltpu.make_async_copy(out_v, out_slice(og), sos[b]).wait()

                for h in range(2):
                    ic = 2 * og + h
                    in_v = ins[h]
                    pltpu.make_async_copy(in_slice(ic), in_v, sis[h]).wait()

                    @plsc.parallel_loop(0, _F // _L, unroll=8)
                    def col(j):
                        pcol = perm_v[pl.ds(j * _L, _L)]
                        for r in range(_CI):
                            rvec = jnp.full((_L,), r, jnp.int32)
                            out_v[_CI * h + r, pl.ds(j * _L, _L)] = (
                                plsc.load_gather(in_v, [rvec, pcol])
                            )

                    @pl.when(ic + 2 < nic)
                    def _next_in():
                        pltpu.async_copy(in_slice(ic + 2), in_v, sis[h])

                pltpu.async_copy(out_v, out_slice(og), sos[b])

        pltpu.make_async_copy(o0, out_slice(0), so0).wait()
        pltpu.make_async_copy(o1, out_slice(1), so1).wait()

    return k


_PERMUTE = _build_permute(4 * 4096)


def kernel(x, perm):
    b, s, f = x.shape
    out = _PERMUTE(x.reshape(b * s, f), perm.astype(jnp.int32))
    return out.reshape(b, s, f)
